# baseline (speedup 1.0000x reference)
import functools

import jax
import jax.numpy as jnp
from jax import lax
from jax.experimental import pallas as pl
from jax.experimental.pallas import tpu as pltpu
from jax.experimental.pallas import tpu_sc as plsc

NC, NS = 2, 16
NW = NC * NS
BATCH, SEQ, D = 4, 2048, 512
WPB = NW // BATCH
BPW = SEQ // WPB
CHUNK = 16
NCHUNK = BPW // CHUNK
NPAIR = NCHUNK // 2
PBUF = 7

_mesh = plsc.VectorSubcoreMesh(core_axis_name="c", subcore_axis_name="s")


@functools.partial(
    pl.kernel,
    mesh=_mesh,
    out_type=jax.ShapeDtypeStruct((BATCH, SEQ, D), jnp.float32),
    scratch_types=[
        pltpu.VMEM((BPW,), jnp.int32),
        *[pltpu.VMEM((2 * CHUNK, D), jnp.float32) for _ in range(PBUF)],
        *[pltpu.SemaphoreType.DMA for _ in range(2 * PBUF + PBUF + 1)],
    ],
)
def _embed_gather(idx_hbm, table_hbm, out_hbm, idx_v, *scratch):
    bufs = scratch[:PBUF]
    gsems = scratch[PBUF:3 * PBUF]
    osems = scratch[3 * PBUF:4 * PBUF]
    isem = scratch[4 * PBUF]

    wid = lax.axis_index("s") * NC + lax.axis_index("c")
    row = wid // WPB
    seq0 = (wid % WPB) * BPW

    HALF = BPW // 2
    ih0 = pltpu.async_copy(
        idx_hbm.at[row, pl.ds(seq0, HALF)], idx_v.at[pl.ds(0, HALF)], isem)
    ih1 = pltpu.async_copy(
        idx_hbm.at[row, pl.ds(seq0 + HALF, HALF)],
        idx_v.at[pl.ds(HALF, HALF)], isem)

    def start_gather(c):
        p = (c // 2) % PBUF
        return pltpu.async_copy(
            table_hbm.at[idx_v.at[pl.ds(c * CHUNK, CHUNK)]],
            bufs[p].at[pl.ds((c % 2) * CHUNK, CHUNK)],
            gsems[c % (2 * PBUF)])

    gh = [None] * NCHUNK
    oh = [None] * NPAIR
    ih0.wait()
    for c in range(2 * min(PBUF, NPAIR)):
        if c * CHUNK == HALF:
            ih1.wait()
            ih1 = None
        gh[c] = start_gather(c)
    if ih1 is not None:
        ih1.wait()

    for p in range(NPAIR):
        b = p % PBUF
        gh[2 * p].wait()
        gh[2 * p + 1].wait()
        oh[p] = pltpu.async_copy(
            bufs[b], out_hbm.at[row, pl.ds(seq0 + p * 2 * CHUNK, 2 * CHUNK)],
            osems[b])
        d = p - 2
        if d >= 0 and d + PBUF < NPAIR:
            oh[d].wait()
            oh[d] = None
            gh[2 * (d + PBUF)] = start_gather(2 * (d + PBUF))
            gh[2 * (d + PBUF) + 1] = start_gather(2 * (d + PBUF) + 1)
    for p in range(NPAIR):
        if oh[p] is not None:
            oh[p].wait()


def kernel(x, table):
    return _embed_gather(x.astype(jnp.int32), table)
